# baseline (device time: 160885 ns/iter reference)
import jax
import jax.numpy as jnp
from jax import lax
from jax.experimental import pallas as pl
from jax.experimental.pallas import tpu as pltpu

N_DEV = 4


def kernel(x, k, Wp):
    B, S, C = x.shape
    K = k.shape[0]
    N = Wp.shape[1]
    M = B * S

    def body(x_ref, k_ref, w_ref, out_ref, comm_ref, send_sems, recv_sems):
        my = lax.axis_index("i")
        right = (my + 1) % N_DEV
        left = (my + N_DEV - 1) % N_DEV

        barrier_sem = pltpu.get_barrier_semaphore()
        for nbr in (left, right):
            pl.semaphore_signal(
                barrier_sem, inc=1,
                device_id=(nbr,), device_id_type=pl.DeviceIdType.MESH,
            )
        pl.semaphore_wait(barrier_sem, 2)

        xv = x_ref[:, :, :]
        acc = xv * k_ref[K - 1, :]
        for t in range(K - 1):
            shift = K - 1 - t
            shifted = jnp.concatenate(
                [jnp.zeros((B, shift, C), dtype=xv.dtype),
                 xv[:, : S - shift, :]],
                axis=1,
            )
            acc = acc + shifted * k_ref[t, :]
        a = acc * (1.0 / (1.0 + jnp.exp(-acc)))
        a2 = a.reshape(M, C).astype(jnp.bfloat16)
        w = w_ref[:, :].astype(jnp.bfloat16)
        partial = jnp.dot(a2, w, preferred_element_type=jnp.float32)

        total = partial
        comm_ref[0, :, :] = partial.astype(jnp.bfloat16)

        for h in range(N_DEV - 1):
            rdma = pltpu.make_async_remote_copy(
                src_ref=comm_ref.at[h],
                dst_ref=comm_ref.at[h + 1],
                send_sem=send_sems.at[h],
                recv_sem=recv_sems.at[h],
                device_id=(right,),
                device_id_type=pl.DeviceIdType.MESH,
            )
            rdma.start()
            rdma.wait()
            total = total + comm_ref[h + 1, :, :].astype(jnp.float32)

        out_ref[:, :, :] = total.reshape(B, S, N)

    return pl.pallas_call(
        body,
        out_shape=jax.ShapeDtypeStruct((B, S, N), jnp.float32),
        in_specs=[
            pl.BlockSpec(memory_space=pltpu.VMEM),
            pl.BlockSpec(memory_space=pltpu.VMEM),
            pl.BlockSpec(memory_space=pltpu.VMEM),
        ],
        out_specs=pl.BlockSpec(memory_space=pltpu.VMEM),
        scratch_shapes=[
            pltpu.VMEM((N_DEV, M, N), jnp.bfloat16),
            pltpu.SemaphoreType.DMA((N_DEV - 1,)),
            pltpu.SemaphoreType.DMA((N_DEV - 1,)),
        ],
        compiler_params=pltpu.CompilerParams(collective_id=0),
    )(x, k, Wp)


# device time: 60541 ns/iter; 2.6575x vs baseline; 2.6575x over previous
import jax
import jax.numpy as jnp
from jax import lax
from jax.experimental import pallas as pl
from jax.experimental.pallas import tpu as pltpu

N_DEV = 4


def kernel(x, k, Wp):
    B, S, C = x.shape
    K = k.shape[0]
    N = Wp.shape[1]
    M = B * S
    H = M // 2
    Q = H // 2
    E = Q // 2

    def body(x_ref, k_ref, w_ref, out_ref, acc_ref, sbuf_ref, rbuf_ref,
             send_sems, recv_sems):
        my = lax.axis_index("i")
        x_b = my // 2
        y_b = (my // 2) ^ (my % 2)
        py = my ^ 1
        px = (N_DEV - 1) - my

        barrier_sem = pltpu.get_barrier_semaphore()
        for nbr in (py, px):
            pl.semaphore_signal(
                barrier_sem, inc=1,
                device_id=(nbr,), device_id_type=pl.DeviceIdType.MESH,
            )
        pl.semaphore_wait(barrier_sem, 2)

        xv = x_ref[:, :, :]
        conv = xv * k_ref[K - 1, :]
        for t in range(K - 1):
            shift = K - 1 - t
            shifted = jnp.concatenate(
                [jnp.zeros((B, shift, C), dtype=xv.dtype),
                 xv[:, : S - shift, :]],
                axis=1,
            )
            conv = conv + shifted * k_ref[t, :]
        a = conv * (1.0 / (1.0 + jnp.exp(-conv)))
        a2 = a.reshape(M, C).astype(jnp.bfloat16)
        w = w_ref[:, :].astype(jnp.bfloat16)
        partial = jnp.dot(a2, w, preferred_element_type=jnp.float32)

        acc_ref[:, :] = partial
        sbuf_ref[:, :] = partial.astype(jnp.bfloat16)

        rdmas = []

        def exchange(src_off, n_rows, dst_slot, sem_idx, partner):
            rdma = pltpu.make_async_remote_copy(
                src_ref=sbuf_ref.at[pl.ds(src_off, n_rows)],
                dst_ref=rbuf_ref.at[pl.ds(dst_slot, n_rows)],
                send_sem=send_sems.at[sem_idx],
                recv_sem=recv_sems.at[sem_idx],
                device_id=(partner,),
                device_id_type=pl.DeviceIdType.MESH,
            )
            rdma.start()
            rdmas.append(rdma)
            return rdma

        def add_rows(off, slot, n_rows):
            acc_ref[pl.ds(off, n_rows), :] = (
                acc_ref[pl.ds(off, n_rows), :]
                + rbuf_ref[pl.ds(slot, n_rows), :].astype(jnp.float32)
            )

        def stage(off, n_rows):
            sbuf_ref[pl.ds(off, n_rows), :] = (
                acc_ref[pl.ds(off, n_rows), :].astype(jnp.bfloat16)
            )

        k0 = y_b * Q
        k1 = H + x_b * Q
        k2_0 = k0 + x_b * E
        k2_1 = k1 + y_b * E

        p1a = exchange((1 - y_b) * Q, Q, 0, 0, py)
        p1b = exchange(H + (1 - x_b) * Q, Q, Q, 1, px)
        p1a.wait_recv()
        p1b.wait_recv()
        add_rows(k0, 0, Q)
        add_rows(k1, Q, Q)

        s2_0 = k0 + (1 - x_b) * E
        s2_1 = k1 + (1 - y_b) * E
        stage(s2_0, E)
        stage(s2_1, E)
        p2a = exchange(s2_0, E, 2 * Q, 2, px)
        p2b = exchange(s2_1, E, 2 * Q + E, 3, py)
        p2a.wait_recv()
        p2b.wait_recv()
        add_rows(k2_0, 2 * Q, E)
        add_rows(k2_1, 2 * Q + E, E)

        stage(k2_0, E)
        stage(k2_1, E)
        p3a = exchange(k2_0, E, 3 * Q, 4, px)
        p3b = exchange(k2_1, E, 3 * Q + E, 5, py)
        p3a.wait_recv()
        p3b.wait_recv()
        acc_ref[pl.ds(k0 + (1 - x_b) * E, E), :] = (
            rbuf_ref[pl.ds(3 * Q, E), :].astype(jnp.float32))
        acc_ref[pl.ds(k1 + (1 - y_b) * E, E), :] = (
            rbuf_ref[pl.ds(3 * Q + E, E), :].astype(jnp.float32))

        stage(k0 + (1 - x_b) * E, E)
        stage(k1 + (1 - y_b) * E, E)
        p4a = exchange(k0, Q, 4 * Q, 6, py)
        p4b = exchange(k1, Q, 5 * Q, 7, px)
        p4a.wait_recv()
        p4b.wait_recv()
        acc_ref[pl.ds((1 - y_b) * Q, Q), :] = (
            rbuf_ref[pl.ds(4 * Q, Q), :].astype(jnp.float32))
        acc_ref[pl.ds(H + (1 - x_b) * Q, Q), :] = (
            rbuf_ref[pl.ds(5 * Q, Q), :].astype(jnp.float32))

        for r in rdmas:
            r.wait_send()

        out_ref[:, :, :] = acc_ref[:, :].reshape(B, S, N)

    return pl.pallas_call(
        body,
        out_shape=jax.ShapeDtypeStruct((B, S, N), jnp.float32),
        in_specs=[
            pl.BlockSpec(memory_space=pltpu.VMEM),
            pl.BlockSpec(memory_space=pltpu.VMEM),
            pl.BlockSpec(memory_space=pltpu.VMEM),
        ],
        out_specs=pl.BlockSpec(memory_space=pltpu.VMEM),
        scratch_shapes=[
            pltpu.VMEM((M, N), jnp.float32),
            pltpu.VMEM((M, N), jnp.bfloat16),
            pltpu.VMEM((6 * M // 4, N), jnp.bfloat16),
            pltpu.SemaphoreType.DMA((8,)),
            pltpu.SemaphoreType.DMA((8,)),
        ],
        compiler_params=pltpu.CompilerParams(collective_id=0),
    )(x, k, Wp)


# device time: 56709 ns/iter; 2.8370x vs baseline; 1.0676x over previous
import jax
import jax.numpy as jnp
from jax import lax
from jax.experimental import pallas as pl
from jax.experimental.pallas import tpu as pltpu

N_DEV = 4


def kernel(x, k, Wp):
    B, S, C = x.shape
    K = k.shape[0]
    N = Wp.shape[1]
    M = B * S
    H = M // 2
    Q = H // 2
    E = Q // 2

    def body(x_ref, k_ref, w_ref, out_ref, acc_ref, sbuf_ref, rbuf_ref,
             send_sems, recv_sems):
        my = lax.axis_index("i")
        x_b = my // 2
        y_b = (my // 2) ^ (my % 2)
        py = my ^ 1
        px = (N_DEV - 1) - my

        barrier_sem = pltpu.get_barrier_semaphore()
        for nbr in (py, px):
            pl.semaphore_signal(
                barrier_sem, inc=1,
                device_id=(nbr,), device_id_type=pl.DeviceIdType.MESH,
            )
        pl.semaphore_wait(barrier_sem, 2)

        w = w_ref[:, :].astype(jnp.bfloat16)

        def partial_batch(b):
            xb = x_ref[pl.ds(b, 1), :, :]
            conv = xb * k_ref[K - 1, :]
            for t in range(K - 1):
                shift = K - 1 - t
                shifted = jnp.concatenate(
                    [jnp.zeros((1, shift, C), dtype=xb.dtype),
                     xb[:, : S - shift, :]],
                    axis=1,
                )
                conv = conv + shifted * k_ref[t, :]
            a = conv * (1.0 / (1.0 + jnp.exp(-conv)))
            a2 = a.reshape(S, C).astype(jnp.bfloat16)
            return jnp.dot(a2, w, preferred_element_type=jnp.float32)

        rdmas = []

        def exchange(src_off, n_rows, dst_slot, sem_idx, partner):
            rdma = pltpu.make_async_remote_copy(
                src_ref=sbuf_ref.at[pl.ds(src_off, n_rows)],
                dst_ref=rbuf_ref.at[pl.ds(dst_slot, n_rows)],
                send_sem=send_sems.at[sem_idx],
                recv_sem=recv_sems.at[sem_idx],
                device_id=(partner,),
                device_id_type=pl.DeviceIdType.MESH,
            )
            rdma.start()
            rdmas.append(rdma)
            return rdma

        def add_rows(off, slot, n_rows):
            acc_ref[pl.ds(off, n_rows), :] = (
                acc_ref[pl.ds(off, n_rows), :]
                + rbuf_ref[pl.ds(slot, n_rows), :].astype(jnp.float32)
            )

        def stage(off, n_rows):
            sbuf_ref[pl.ds(off, n_rows), :] = (
                acc_ref[pl.ds(off, n_rows), :].astype(jnp.bfloat16)
            )

        k0 = y_b * Q
        k1 = H + x_b * Q
        k2_0 = k0 + x_b * E
        k2_1 = k1 + y_b * E

        s1_0 = (1 - y_b) * Q
        s1_1 = H + (1 - x_b) * Q
        sbuf_ref[pl.ds(s1_0, Q), :] = partial_batch(1 - y_b).astype(
            jnp.bfloat16)
        sbuf_ref[pl.ds(s1_1, Q), :] = partial_batch(2 + (1 - x_b)).astype(
            jnp.bfloat16)

        p1a = exchange(s1_0, Q, 0, 0, py)
        p1b = exchange(s1_1, Q, Q, 1, px)

        acc_ref[pl.ds(k0, Q), :] = partial_batch(y_b)
        acc_ref[pl.ds(k1, Q), :] = partial_batch(2 + x_b)

        p1a.wait_recv()
        p1b.wait_recv()
        add_rows(k0, 0, Q)
        add_rows(k1, Q, Q)

        s2_0 = k0 + (1 - x_b) * E
        s2_1 = k1 + (1 - y_b) * E
        stage(s2_0, E)
        stage(s2_1, E)
        p2a = exchange(s2_0, E, 2 * Q, 2, px)
        p2b = exchange(s2_1, E, 2 * Q + E, 3, py)
        p2a.wait_recv()
        p2b.wait_recv()
        add_rows(k2_0, 2 * Q, E)
        add_rows(k2_1, 2 * Q + E, E)

        stage(k2_0, E)
        stage(k2_1, E)
        p3a = exchange(k2_0, E, 3 * Q, 4, px)
        p3b = exchange(k2_1, E, 3 * Q + E, 5, py)
        p3a.wait_recv()
        p3b.wait_recv()
        acc_ref[pl.ds(k0 + (1 - x_b) * E, E), :] = (
            rbuf_ref[pl.ds(3 * Q, E), :].astype(jnp.float32))
        acc_ref[pl.ds(k1 + (1 - y_b) * E, E), :] = (
            rbuf_ref[pl.ds(3 * Q + E, E), :].astype(jnp.float32))

        stage(k0 + (1 - x_b) * E, E)
        stage(k1 + (1 - y_b) * E, E)
        p4a = exchange(k0, Q, 4 * Q, 6, py)
        p4b = exchange(k1, Q, 5 * Q, 7, px)

        out_ref[pl.ds(y_b, 1), :, :] = (
            acc_ref[pl.ds(k0, Q), :].reshape(1, S, N))
        out_ref[pl.ds(2 + x_b, 1), :, :] = (
            acc_ref[pl.ds(k1, Q), :].reshape(1, S, N))

        p4a.wait_recv()
        p4b.wait_recv()
        out_ref[pl.ds(1 - y_b, 1), :, :] = (
            rbuf_ref[pl.ds(4 * Q, Q), :].astype(jnp.float32).reshape(1, S, N))
        out_ref[pl.ds(2 + (1 - x_b), 1), :, :] = (
            rbuf_ref[pl.ds(5 * Q, Q), :].astype(jnp.float32).reshape(1, S, N))

        for r in rdmas:
            r.wait_send()

    return pl.pallas_call(
        body,
        out_shape=jax.ShapeDtypeStruct((B, S, N), jnp.float32),
        in_specs=[
            pl.BlockSpec(memory_space=pltpu.VMEM),
            pl.BlockSpec(memory_space=pltpu.VMEM),
            pl.BlockSpec(memory_space=pltpu.VMEM),
        ],
        out_specs=pl.BlockSpec(memory_space=pltpu.VMEM),
        scratch_shapes=[
            pltpu.VMEM((M, N), jnp.float32),
            pltpu.VMEM((M, N), jnp.bfloat16),
            pltpu.VMEM((6 * M // 4, N), jnp.bfloat16),
            pltpu.SemaphoreType.DMA((8,)),
            pltpu.SemaphoreType.DMA((8,)),
        ],
        compiler_params=pltpu.CompilerParams(collective_id=0),
    )(x, k, Wp)


# device time: 56584 ns/iter; 2.8433x vs baseline; 1.0022x over previous
import jax
import jax.numpy as jnp
from jax import lax
from jax.experimental import pallas as pl
from jax.experimental.pallas import tpu as pltpu

N_DEV = 4


def kernel(x, k, Wp):
    B, S, C = x.shape
    K = k.shape[0]
    N = Wp.shape[1]
    M = B * S
    H = M // 2
    Q = H // 2
    E = Q // 2

    def body(x_ref, k_ref, w_ref, out_ref, acc_ref, sbuf_ref, rbuf_ref,
             send_sems, recv_sems):
        my = lax.axis_index("i")
        x_b = my // 2
        y_b = (my // 2) ^ (my % 2)
        py = my ^ 1
        px = (N_DEV - 1) - my

        barrier_sem = pltpu.get_barrier_semaphore()
        for nbr in (py, px):
            pl.semaphore_signal(
                barrier_sem, inc=1,
                device_id=(nbr,), device_id_type=pl.DeviceIdType.MESH,
            )
        pl.semaphore_wait(barrier_sem, 2)

        w = w_ref[:, :].astype(jnp.bfloat16)

        kb = k_ref[:, :].astype(jnp.bfloat16)

        def partial_batch(b):
            xb = x_ref[pl.ds(b, 1), :, :].astype(jnp.bfloat16)
            conv = xb * kb[K - 1, :]
            for t in range(K - 1):
                shift = K - 1 - t
                shifted = jnp.concatenate(
                    [jnp.zeros((1, shift, C), dtype=xb.dtype),
                     xb[:, : S - shift, :]],
                    axis=1,
                )
                conv = conv + shifted * kb[t, :]
            cf = conv.astype(jnp.float32)
            a = cf * (1.0 / (1.0 + jnp.exp(-cf)))
            a2 = a.reshape(S, C).astype(jnp.bfloat16)
            return jnp.dot(a2, w, preferred_element_type=jnp.float32)

        rdmas = []

        def exchange(src_off, n_rows, dst_slot, sem_idx, partner):
            rdma = pltpu.make_async_remote_copy(
                src_ref=sbuf_ref.at[pl.ds(src_off, n_rows)],
                dst_ref=rbuf_ref.at[pl.ds(dst_slot, n_rows)],
                send_sem=send_sems.at[sem_idx],
                recv_sem=recv_sems.at[sem_idx],
                device_id=(partner,),
                device_id_type=pl.DeviceIdType.MESH,
            )
            rdma.start()
            rdmas.append(rdma)
            return rdma

        def add_rows(off, slot, n_rows):
            acc_ref[pl.ds(off, n_rows), :] = (
                acc_ref[pl.ds(off, n_rows), :]
                + rbuf_ref[pl.ds(slot, n_rows), :].astype(jnp.float32)
            )

        def stage(off, n_rows):
            sbuf_ref[pl.ds(off, n_rows), :] = (
                acc_ref[pl.ds(off, n_rows), :].astype(jnp.bfloat16)
            )

        k0 = y_b * Q
        k1 = H + x_b * Q
        k2_0 = k0 + x_b * E
        k2_1 = k1 + y_b * E

        s1_0 = (1 - y_b) * Q
        s1_1 = H + (1 - x_b) * Q
        sbuf_ref[pl.ds(s1_0, Q), :] = partial_batch(1 - y_b).astype(
            jnp.bfloat16)
        p1a = exchange(s1_0, Q, 0, 0, py)
        sbuf_ref[pl.ds(s1_1, Q), :] = partial_batch(2 + (1 - x_b)).astype(
            jnp.bfloat16)
        p1b = exchange(s1_1, Q, Q, 1, px)

        acc_ref[pl.ds(k0, Q), :] = partial_batch(y_b)
        acc_ref[pl.ds(k1, Q), :] = partial_batch(2 + x_b)

        p1a.wait_recv()
        p1b.wait_recv()
        add_rows(k0, 0, Q)
        add_rows(k1, Q, Q)

        s2_0 = k0 + (1 - x_b) * E
        s2_1 = k1 + (1 - y_b) * E
        stage(s2_0, E)
        stage(s2_1, E)
        p2a = exchange(s2_0, E, 2 * Q, 2, px)
        p2b = exchange(s2_1, E, 2 * Q + E, 3, py)
        p2a.wait_recv()
        p2b.wait_recv()
        add_rows(k2_0, 2 * Q, E)
        add_rows(k2_1, 2 * Q + E, E)

        stage(k2_0, E)
        stage(k2_1, E)
        p3a = exchange(k2_0, E, 3 * Q, 4, px)
        p3b = exchange(k2_1, E, 3 * Q + E, 5, py)
        p3a.wait_recv()
        p3b.wait_recv()
        acc_ref[pl.ds(k0 + (1 - x_b) * E, E), :] = (
            rbuf_ref[pl.ds(3 * Q, E), :].astype(jnp.float32))
        acc_ref[pl.ds(k1 + (1 - y_b) * E, E), :] = (
            rbuf_ref[pl.ds(3 * Q + E, E), :].astype(jnp.float32))

        stage(k0 + (1 - x_b) * E, E)
        stage(k1 + (1 - y_b) * E, E)
        p4a = exchange(k0, Q, 4 * Q, 6, py)
        p4b = exchange(k1, Q, 5 * Q, 7, px)

        out_ref[pl.ds(y_b, 1), :, :] = (
            acc_ref[pl.ds(k0, Q), :].reshape(1, S, N))
        out_ref[pl.ds(2 + x_b, 1), :, :] = (
            acc_ref[pl.ds(k1, Q), :].reshape(1, S, N))

        p4a.wait_recv()
        p4b.wait_recv()
        out_ref[pl.ds(1 - y_b, 1), :, :] = (
            rbuf_ref[pl.ds(4 * Q, Q), :].astype(jnp.float32).reshape(1, S, N))
        out_ref[pl.ds(2 + (1 - x_b), 1), :, :] = (
            rbuf_ref[pl.ds(5 * Q, Q), :].astype(jnp.float32).reshape(1, S, N))

        for r in rdmas:
            r.wait_send()

    return pl.pallas_call(
        body,
        out_shape=jax.ShapeDtypeStruct((B, S, N), jnp.float32),
        in_specs=[
            pl.BlockSpec(memory_space=pltpu.VMEM),
            pl.BlockSpec(memory_space=pltpu.VMEM),
            pl.BlockSpec(memory_space=pltpu.VMEM),
        ],
        out_specs=pl.BlockSpec(memory_space=pltpu.VMEM),
        scratch_shapes=[
            pltpu.VMEM((M, N), jnp.float32),
            pltpu.VMEM((M, N), jnp.bfloat16),
            pltpu.VMEM((6 * M // 4, N), jnp.bfloat16),
            pltpu.SemaphoreType.DMA((8,)),
            pltpu.SemaphoreType.DMA((8,)),
        ],
        compiler_params=pltpu.CompilerParams(collective_id=0),
    )(x, k, Wp)


# device time: 54714 ns/iter; 2.9405x vs baseline; 1.0342x over previous
import jax
import jax.numpy as jnp
from jax import lax
from jax.experimental import pallas as pl
from jax.experimental.pallas import tpu as pltpu

N_DEV = 4


def kernel(x, k, Wp):
    B, S, C = x.shape
    K = k.shape[0]
    N = Wp.shape[1]
    M = B * S
    H = M // 2
    Q = H // 2
    E = Q // 2

    def body(x_ref, k_ref, w_ref, out_ref, acc_ref, sbuf_ref, rbuf_ref,
             send_sems, recv_sems):
        my = lax.axis_index("i")
        x_b = my // 2
        y_b = (my // 2) ^ (my % 2)
        py = my ^ 1
        px = (N_DEV - 1) - my

        barrier_sem = pltpu.get_barrier_semaphore()
        for nbr in (py, px):
            pl.semaphore_signal(
                barrier_sem, inc=1,
                device_id=(nbr,), device_id_type=pl.DeviceIdType.MESH,
            )
        pl.semaphore_wait(barrier_sem, 2)

        w = w_ref[:, :].astype(jnp.bfloat16)
        kb = k_ref[:, :].astype(jnp.bfloat16)

        def partial_batch(b):
            xb = x_ref[pl.ds(b, 1), :, :].astype(jnp.bfloat16)
            conv = xb * kb[K - 1, :]
            for t in range(K - 1):
                shift = K - 1 - t
                shifted = jnp.concatenate(
                    [jnp.zeros((1, shift, C), dtype=xb.dtype),
                     xb[:, : S - shift, :]],
                    axis=1,
                )
                conv = conv + shifted * kb[t, :]
            cf = conv.astype(jnp.float32)
            a = cf * (1.0 / (1.0 + jnp.exp(-cf)))
            a2 = a.reshape(S, C).astype(jnp.bfloat16)
            return jnp.dot(a2, w, preferred_element_type=jnp.float32)

        rdmas = []

        def exchange(src_ref, dst_ref, sem_idx, partner):
            rdma = pltpu.make_async_remote_copy(
                src_ref=src_ref,
                dst_ref=dst_ref,
                send_sem=send_sems.at[sem_idx],
                recv_sem=recv_sems.at[sem_idx],
                device_id=(partner,),
                device_id_type=pl.DeviceIdType.MESH,
            )
            rdma.start()
            rdmas.append(rdma)
            return rdma

        def add_rows(off, slot, n_rows):
            acc_ref[pl.ds(off, n_rows), :] = (
                acc_ref[pl.ds(off, n_rows), :]
                + rbuf_ref[pl.ds(slot, n_rows), :].astype(jnp.float32)
            )

        k0 = y_b * Q
        k1 = H + x_b * Q
        k2_0 = k0 + x_b * E
        k2_1 = k1 + y_b * E

        s1_0 = (1 - y_b) * Q
        s1_1 = H + (1 - x_b) * Q
        sbuf_ref[pl.ds(s1_0, Q), :] = partial_batch(1 - y_b).astype(
            jnp.bfloat16)
        p1a = exchange(sbuf_ref.at[pl.ds(s1_0, Q)],
                       rbuf_ref.at[pl.ds(0, Q)], 0, py)
        sbuf_ref[pl.ds(s1_1, Q), :] = partial_batch(2 + (1 - x_b)).astype(
            jnp.bfloat16)
        p1b = exchange(sbuf_ref.at[pl.ds(s1_1, Q)],
                       rbuf_ref.at[pl.ds(Q, Q)], 1, px)

        acc_ref[pl.ds(k0, Q), :] = partial_batch(y_b)
        acc_ref[pl.ds(k1, Q), :] = partial_batch(2 + x_b)

        p1a.wait_recv()
        p1b.wait_recv()
        add_rows(k0, 0, Q)
        add_rows(k1, Q, Q)

        s2_0 = k0 + (1 - x_b) * E
        s2_1 = k1 + (1 - y_b) * E
        sbuf_ref[pl.ds(s2_0, E), :] = (
            acc_ref[pl.ds(s2_0, E), :].astype(jnp.bfloat16))
        sbuf_ref[pl.ds(s2_1, E), :] = (
            acc_ref[pl.ds(s2_1, E), :].astype(jnp.bfloat16))
        p2a = exchange(sbuf_ref.at[pl.ds(s2_0, E)],
                       rbuf_ref.at[pl.ds(2 * Q, E)], 2, px)
        p2b = exchange(sbuf_ref.at[pl.ds(s2_1, E)],
                       rbuf_ref.at[pl.ds(2 * Q + E, E)], 3, py)
        p2a.wait_recv()
        p2b.wait_recv()
        add_rows(k2_0, 2 * Q, E)
        add_rows(k2_1, 2 * Q + E, E)

        out_ref[pl.ds(y_b, 1), pl.ds(x_b * E, E), :] = (
            acc_ref[pl.ds(k2_0, E), :].astype(jnp.bfloat16).reshape(1, E, N))
        out_ref[pl.ds(2 + x_b, 1), pl.ds(y_b * E, E), :] = (
            acc_ref[pl.ds(k2_1, E), :].astype(jnp.bfloat16).reshape(1, E, N))

        o3a = out_ref.at[pl.ds(y_b, 1), pl.ds(x_b * E, E)]
        o3b = out_ref.at[pl.ds(2 + x_b, 1), pl.ds(y_b * E, E)]
        p3a = exchange(o3a, o3a, 4, px)
        p3b = exchange(o3b, o3b, 5, py)
        p3a.wait_recv()
        p3b.wait_recv()

        o4a = out_ref.at[pl.ds(y_b, 1)]
        o4b = out_ref.at[pl.ds(2 + x_b, 1)]
        p4a = exchange(o4a, o4a, 6, py)
        p4b = exchange(o4b, o4b, 7, px)
        p4a.wait_recv()
        p4b.wait_recv()

        for r in rdmas:
            r.wait_send()

    return pl.pallas_call(
        body,
        out_shape=jax.ShapeDtypeStruct((B, S, N), jnp.bfloat16),
        in_specs=[
            pl.BlockSpec(memory_space=pltpu.VMEM),
            pl.BlockSpec(memory_space=pltpu.VMEM),
            pl.BlockSpec(memory_space=pltpu.VMEM),
        ],
        out_specs=pl.BlockSpec(memory_space=pltpu.VMEM),
        scratch_shapes=[
            pltpu.VMEM((M, N), jnp.float32),
            pltpu.VMEM((M, N), jnp.bfloat16),
            pltpu.VMEM((3 * M // 4, N), jnp.bfloat16),
            pltpu.SemaphoreType.DMA((8,)),
            pltpu.SemaphoreType.DMA((8,)),
        ],
        compiler_params=pltpu.CompilerParams(collective_id=0),
    )(x, k, Wp)


# device time: 52582 ns/iter; 3.0597x vs baseline; 1.0405x over previous
import jax
import jax.numpy as jnp
from jax import lax
from jax.experimental import pallas as pl
from jax.experimental.pallas import tpu as pltpu

N_DEV = 4


def kernel(x, k, Wp):
    B, S, C = x.shape
    K = k.shape[0]
    N = Wp.shape[1]
    M = B * S
    H = M // 2
    Q = H // 2
    E = Q // 2

    def body(x_ref, k_ref, w_ref, out_ref, acc_ref, sbuf_ref, rbuf_ref,
             send_sems, recv_sems):
        my = lax.axis_index("i")
        x_b = my // 2
        y_b = (my // 2) ^ (my % 2)
        py = my ^ 1
        px = (N_DEV - 1) - my

        barrier_sem = pltpu.get_barrier_semaphore()
        for nbr in (py, px):
            pl.semaphore_signal(
                barrier_sem, inc=1,
                device_id=(nbr,), device_id_type=pl.DeviceIdType.MESH,
            )
        pl.semaphore_wait(barrier_sem, 2)

        w = w_ref[:, :].astype(jnp.bfloat16)
        kb = k_ref[:, :].astype(jnp.bfloat16)

        def partial_batch(b):
            xb = x_ref[pl.ds(b, 1), :, :].astype(jnp.bfloat16)
            conv = xb * kb[K - 1, :]
            for t in range(K - 1):
                shift = K - 1 - t
                shifted = jnp.concatenate(
                    [jnp.zeros((1, shift, C), dtype=xb.dtype),
                     xb[:, : S - shift, :]],
                    axis=1,
                )
                conv = conv + shifted * kb[t, :]
            cf = conv.astype(jnp.float32)
            a = cf * (1.0 / (1.0 + jnp.exp(-cf)))
            a2 = a.reshape(S, C).astype(jnp.bfloat16)
            return jnp.dot(a2, w, preferred_element_type=jnp.float32)

        rdmas = []

        def exchange(src_ref, dst_ref, sem_idx, partner):
            rdma = pltpu.make_async_remote_copy(
                src_ref=src_ref,
                dst_ref=dst_ref,
                send_sem=send_sems.at[sem_idx],
                recv_sem=recv_sems.at[sem_idx],
                device_id=(partner,),
                device_id_type=pl.DeviceIdType.MESH,
            )
            rdma.start()
            rdmas.append(rdma)
            return rdma

        k0 = y_b * Q
        k1 = H + x_b * Q
        s1_0 = (1 - y_b) * Q
        s1_1 = H + (1 - x_b) * Q
        s2_0 = k0 + (1 - x_b) * E
        s2_1 = k1 + (1 - y_b) * E
        k2_0 = k0 + x_b * E
        k2_1 = k1 + y_b * E

        c1a = (1 - x_b) * E
        c2a = x_b * E
        c1b = (1 - y_b) * E
        c2b = y_b * E

        sbuf_ref[pl.ds(s1_0, Q), :] = partial_batch(1 - y_b).astype(
            jnp.bfloat16)
        p1a_c1 = exchange(sbuf_ref.at[pl.ds(s1_0 + c1a, E)],
                          rbuf_ref.at[pl.ds(c1a, E)], 0, py)
        p1a_c2 = exchange(sbuf_ref.at[pl.ds(s1_0 + c2a, E)],
                          rbuf_ref.at[pl.ds(c2a, E)], 2, py)
        sbuf_ref[pl.ds(s1_1, Q), :] = partial_batch(2 + (1 - x_b)).astype(
            jnp.bfloat16)
        p1b_c1 = exchange(sbuf_ref.at[pl.ds(s1_1 + c1b, E)],
                          rbuf_ref.at[pl.ds(Q + c1b, E)], 1, px)
        p1b_c2 = exchange(sbuf_ref.at[pl.ds(s1_1 + c2b, E)],
                          rbuf_ref.at[pl.ds(Q + c2b, E)], 3, px)

        acc_ref[pl.ds(k0, Q), :] = partial_batch(y_b)
        acc_ref[pl.ds(k1, Q), :] = partial_batch(2 + x_b)

        p1a_c1.wait_recv()
        sbuf_ref[pl.ds(s2_0, E), :] = (
            acc_ref[pl.ds(s2_0, E), :]
            + rbuf_ref[pl.ds(c1a, E), :].astype(jnp.float32)
        ).astype(jnp.bfloat16)
        p2a = exchange(sbuf_ref.at[pl.ds(s2_0, E)],
                       rbuf_ref.at[pl.ds(2 * Q, E)], 4, px)
        p1b_c1.wait_recv()
        sbuf_ref[pl.ds(s2_1, E), :] = (
            acc_ref[pl.ds(s2_1, E), :]
            + rbuf_ref[pl.ds(Q + c1b, E), :].astype(jnp.float32)
        ).astype(jnp.bfloat16)
        p2b = exchange(sbuf_ref.at[pl.ds(s2_1, E)],
                       rbuf_ref.at[pl.ds(2 * Q + E, E)], 5, py)

        p1a_c2.wait_recv()
        acc_ref[pl.ds(k2_0, E), :] = (
            acc_ref[pl.ds(k2_0, E), :]
            + rbuf_ref[pl.ds(c2a, E), :].astype(jnp.float32))
        p1b_c2.wait_recv()
        acc_ref[pl.ds(k2_1, E), :] = (
            acc_ref[pl.ds(k2_1, E), :]
            + rbuf_ref[pl.ds(Q + c2b, E), :].astype(jnp.float32))

        p2a.wait_recv()
        out_ref[pl.ds(y_b, 1), pl.ds(x_b * E, E), :] = (
            acc_ref[pl.ds(k2_0, E), :]
            + rbuf_ref[pl.ds(2 * Q, E), :].astype(jnp.float32)
        ).astype(jnp.bfloat16).reshape(1, E, N)
        o3a = out_ref.at[pl.ds(y_b, 1), pl.ds(x_b * E, E)]
        p4a_c1 = exchange(o3a, o3a, 8, py)
        p3a = exchange(o3a, o3a, 6, px)

        p2b.wait_recv()
        out_ref[pl.ds(2 + x_b, 1), pl.ds(y_b * E, E), :] = (
            acc_ref[pl.ds(k2_1, E), :]
            + rbuf_ref[pl.ds(2 * Q + E, E), :].astype(jnp.float32)
        ).astype(jnp.bfloat16).reshape(1, E, N)
        o3b = out_ref.at[pl.ds(2 + x_b, 1), pl.ds(y_b * E, E)]
        p4b_c1 = exchange(o3b, o3b, 9, px)
        p3b = exchange(o3b, o3b, 7, py)

        p3a.wait_recv()
        o4a2 = out_ref.at[pl.ds(y_b, 1), pl.ds((1 - x_b) * E, E)]
        p4a_c2 = exchange(o4a2, o4a2, 10, py)
        p3b.wait_recv()
        o4b2 = out_ref.at[pl.ds(2 + x_b, 1), pl.ds((1 - y_b) * E, E)]
        p4b_c2 = exchange(o4b2, o4b2, 11, px)

        p4a_c1.wait_recv()
        p4b_c1.wait_recv()
        p4a_c2.wait_recv()
        p4b_c2.wait_recv()

        for r in rdmas:
            r.wait_send()

    return pl.pallas_call(
        body,
        out_shape=jax.ShapeDtypeStruct((B, S, N), jnp.bfloat16),
        in_specs=[
            pl.BlockSpec(memory_space=pltpu.VMEM),
            pl.BlockSpec(memory_space=pltpu.VMEM),
            pl.BlockSpec(memory_space=pltpu.VMEM),
        ],
        out_specs=pl.BlockSpec(memory_space=pltpu.VMEM),
        scratch_shapes=[
            pltpu.VMEM((M, N), jnp.float32),
            pltpu.VMEM((M, N), jnp.bfloat16),
            pltpu.VMEM((3 * M // 4, N), jnp.bfloat16),
            pltpu.SemaphoreType.DMA((12,)),
            pltpu.SemaphoreType.DMA((12,)),
        ],
        compiler_params=pltpu.CompilerParams(collective_id=0),
    )(x, k, Wp)


# device time: 50648 ns/iter; 3.1765x vs baseline; 1.0382x over previous
import jax
import jax.numpy as jnp
from jax import lax
from jax.experimental import pallas as pl
from jax.experimental.pallas import tpu as pltpu

N_DEV = 4


def kernel(x, k, Wp):
    B, S, C = x.shape
    K = k.shape[0]
    N = Wp.shape[1]
    M = B * S
    H = M // 2
    Q = H // 2
    E = Q // 2

    def body(x_ref, k_ref, w_ref, out_ref, acc_ref, sbuf_ref, rbuf_ref,
             send_sems, recv_sems):
        my = lax.axis_index("i")
        x_b = my // 2
        y_b = (my // 2) ^ (my % 2)
        py = my ^ 1
        px = (N_DEV - 1) - my

        barrier_sem = pltpu.get_barrier_semaphore()
        for nbr in (py, px):
            pl.semaphore_signal(
                barrier_sem, inc=1,
                device_id=(nbr,), device_id_type=pl.DeviceIdType.MESH,
            )
        pl.semaphore_wait(barrier_sem, 2)

        w = w_ref[:, :].astype(jnp.bfloat16)
        kb = k_ref[:, :].astype(jnp.bfloat16)

        def partial_batch(b):
            xb = x_ref[pl.ds(b, 1), :, :].astype(jnp.bfloat16)
            conv = xb * kb[K - 1, :]
            for t in range(K - 1):
                shift = K - 1 - t
                shifted = jnp.concatenate(
                    [jnp.zeros((1, shift, C), dtype=xb.dtype),
                     xb[:, : S - shift, :]],
                    axis=1,
                )
                conv = conv + shifted * kb[t, :]
            cf = conv.astype(jnp.float32)
            a = cf * (1.0 / (1.0 + jnp.exp(-cf)))
            a2 = a.reshape(S, C).astype(jnp.bfloat16)
            return jnp.dot(a2, w, preferred_element_type=jnp.float32)

        rdmas = []

        def exchange(src_ref, dst_ref, sem_idx, partner):
            rdma = pltpu.make_async_remote_copy(
                src_ref=src_ref,
                dst_ref=dst_ref,
                send_sem=send_sems.at[sem_idx],
                recv_sem=recv_sems.at[sem_idx],
                device_id=(partner,),
                device_id_type=pl.DeviceIdType.MESH,
            )
            rdma.start()
            rdmas.append(rdma)
            return rdma

        k0 = y_b * Q
        k1 = H + x_b * Q
        s1_0 = (1 - y_b) * Q
        s1_1 = H + (1 - x_b) * Q
        s2_0 = k0 + (1 - x_b) * E
        s2_1 = k1 + (1 - y_b) * E
        k2_0 = k0 + x_b * E
        k2_1 = k1 + y_b * E

        c1a = (1 - x_b) * E
        c2a = x_b * E
        c1b = (1 - y_b) * E
        c2b = y_b * E

        sbuf_ref[pl.ds(s1_0, Q), :] = partial_batch(1 - y_b).astype(
            jnp.bfloat16)
        p1a_c1 = exchange(sbuf_ref.at[pl.ds(s1_0 + c1a, E)],
                          rbuf_ref.at[pl.ds(c1a, E)], 0, py)
        p1a_c2 = exchange(sbuf_ref.at[pl.ds(s1_0 + c2a, E)],
                          rbuf_ref.at[pl.ds(c2a, E)], 2, py)
        sbuf_ref[pl.ds(s1_1, Q), :] = partial_batch(2 + (1 - x_b)).astype(
            jnp.bfloat16)
        p1b_c1 = exchange(sbuf_ref.at[pl.ds(s1_1 + c1b, E)],
                          rbuf_ref.at[pl.ds(Q + c1b, E)], 1, px)
        p1b_c2 = exchange(sbuf_ref.at[pl.ds(s1_1 + c2b, E)],
                          rbuf_ref.at[pl.ds(Q + c2b, E)], 3, px)

        acc_ref[pl.ds(k0, Q), :] = partial_batch(y_b)
        acc_ref[pl.ds(k1, Q), :] = partial_batch(2 + x_b)

        p1a_c1.wait_recv()
        sbuf_ref[pl.ds(s2_0, E), :] = (
            acc_ref[pl.ds(s2_0, E), :]
            + rbuf_ref[pl.ds(c1a, E), :].astype(jnp.float32)
        ).astype(jnp.bfloat16)
        p2a = exchange(sbuf_ref.at[pl.ds(s2_0, E)],
                       rbuf_ref.at[pl.ds(2 * Q, E)], 4, px)
        p1b_c1.wait_recv()
        sbuf_ref[pl.ds(s2_1, E), :] = (
            acc_ref[pl.ds(s2_1, E), :]
            + rbuf_ref[pl.ds(Q + c1b, E), :].astype(jnp.float32)
        ).astype(jnp.bfloat16)
        p2b = exchange(sbuf_ref.at[pl.ds(s2_1, E)],
                       rbuf_ref.at[pl.ds(2 * Q + E, E)], 5, py)

        p1a_c2.wait_recv()
        acc_ref[pl.ds(k2_0, E), :] = (
            acc_ref[pl.ds(k2_0, E), :]
            + rbuf_ref[pl.ds(c2a, E), :].astype(jnp.float32))
        p1b_c2.wait_recv()
        acc_ref[pl.ds(k2_1, E), :] = (
            acc_ref[pl.ds(k2_1, E), :]
            + rbuf_ref[pl.ds(Q + c2b, E), :].astype(jnp.float32))

        p2a.wait_recv()
        out_ref[pl.ds(y_b, 1), pl.ds(x_b * E, E), :] = (
            acc_ref[pl.ds(k2_0, E), :]
            + rbuf_ref[pl.ds(2 * Q, E), :].astype(jnp.float32)
        ).astype(jnp.bfloat16).reshape(1, E, N)
        o3a = out_ref.at[pl.ds(y_b, 1), pl.ds(x_b * E, E)]
        p3a = exchange(o3a, o3a, 6, px)

        p2b.wait_recv()
        out_ref[pl.ds(2 + x_b, 1), pl.ds(y_b * E, E), :] = (
            acc_ref[pl.ds(k2_1, E), :]
            + rbuf_ref[pl.ds(2 * Q + E, E), :].astype(jnp.float32)
        ).astype(jnp.bfloat16).reshape(1, E, N)
        o3b = out_ref.at[pl.ds(2 + x_b, 1), pl.ds(y_b * E, E)]
        p3b = exchange(o3b, o3b, 7, py)

        p4a_c1 = exchange(o3a, o3a, 8, py)
        p4b_c1 = exchange(o3b, o3b, 9, px)

        p3a.wait_recv()
        o4a2 = out_ref.at[pl.ds(y_b, 1), pl.ds((1 - x_b) * E, E)]
        p4a_c2 = exchange(o4a2, o4a2, 10, py)
        p3b.wait_recv()
        o4b2 = out_ref.at[pl.ds(2 + x_b, 1), pl.ds((1 - y_b) * E, E)]
        p4b_c2 = exchange(o4b2, o4b2, 11, px)

        p4a_c1.wait_recv()
        p4b_c1.wait_recv()
        p4a_c2.wait_recv()
        p4b_c2.wait_recv()

        for r in rdmas:
            r.wait_send()

    return pl.pallas_call(
        body,
        out_shape=jax.ShapeDtypeStruct((B, S, N), jnp.bfloat16),
        in_specs=[
            pl.BlockSpec(memory_space=pltpu.VMEM),
            pl.BlockSpec(memory_space=pltpu.VMEM),
            pl.BlockSpec(memory_space=pltpu.VMEM),
        ],
        out_specs=pl.BlockSpec(memory_space=pltpu.VMEM),
        scratch_shapes=[
            pltpu.VMEM((M, N), jnp.float32),
            pltpu.VMEM((M, N), jnp.bfloat16),
            pltpu.VMEM((3 * M // 4, N), jnp.bfloat16),
            pltpu.SemaphoreType.DMA((12,)),
            pltpu.SemaphoreType.DMA((12,)),
        ],
        compiler_params=pltpu.CompilerParams(collective_id=0),
    )(x, k, Wp)


# device time: 49354 ns/iter; 3.2598x vs baseline; 1.0262x over previous
import jax
import jax.numpy as jnp
from jax import lax
from jax.experimental import pallas as pl
from jax.experimental.pallas import tpu as pltpu

N_DEV = 4


def kernel(x, k, Wp):
    B, S, C = x.shape
    K = k.shape[0]
    N = Wp.shape[1]
    M = B * S
    H = M // 2
    Q = H // 2
    E = Q // 2
    F = E // 2

    def body(x_ref, k_ref, w_ref, out_ref, acc_ref, sbuf_ref, rbuf_ref,
             send_sems, recv_sems):
        my = lax.axis_index("i")
        x_b = my // 2
        y_b = (my // 2) ^ (my % 2)
        py = my ^ 1
        px = (N_DEV - 1) - my

        barrier_sem = pltpu.get_barrier_semaphore()
        for nbr in (py, px):
            pl.semaphore_signal(
                barrier_sem, inc=1,
                device_id=(nbr,), device_id_type=pl.DeviceIdType.MESH,
            )
        pl.semaphore_wait(barrier_sem, 2)

        w = w_ref[:, :].astype(jnp.bfloat16)
        kb = k_ref[:, :].astype(jnp.bfloat16)

        def partial_batch(b):
            xb = x_ref[pl.ds(b, 1), :, :].astype(jnp.bfloat16)
            conv = xb * kb[K - 1, :]
            for t in range(K - 1):
                shift = K - 1 - t
                shifted = jnp.concatenate(
                    [jnp.zeros((1, shift, C), dtype=xb.dtype),
                     xb[:, : S - shift, :]],
                    axis=1,
                )
                conv = conv + shifted * kb[t, :]
            cf = conv.astype(jnp.float32)
            a = cf * (1.0 / (1.0 + jnp.exp(-cf)))
            a2 = a.reshape(S, C).astype(jnp.bfloat16)
            return jnp.dot(a2, w, preferred_element_type=jnp.float32)

        rdmas = []

        def exchange(src_ref, dst_ref, sem_idx, partner):
            rdma = pltpu.make_async_remote_copy(
                src_ref=src_ref,
                dst_ref=dst_ref,
                send_sem=send_sems.at[sem_idx],
                recv_sem=recv_sems.at[sem_idx],
                device_id=(partner,),
                device_id_type=pl.DeviceIdType.MESH,
            )
            rdma.start()
            rdmas.append(rdma)
            return rdma

        k0 = y_b * Q
        k1 = H + x_b * Q
        s1_0 = (1 - y_b) * Q
        s1_1 = H + (1 - x_b) * Q
        s2_0 = k0 + (1 - x_b) * E
        s2_1 = k1 + (1 - y_b) * E
        k2_0 = k0 + x_b * E
        k2_1 = k1 + y_b * E

        c1a = (1 - x_b) * E
        c2a = x_b * E
        c1b = (1 - y_b) * E
        c2b = y_b * E

        sbuf_ref[pl.ds(s1_0, Q), :] = partial_batch(1 - y_b).astype(
            jnp.bfloat16)
        p1a_1 = exchange(sbuf_ref.at[pl.ds(s1_0 + c1a, F)],
                         rbuf_ref.at[pl.ds(c1a, F)], 0, py)
        p1a_2 = exchange(sbuf_ref.at[pl.ds(s1_0 + c1a + F, F)],
                         rbuf_ref.at[pl.ds(c1a + F, F)], 2, py)
        p1a_3 = exchange(sbuf_ref.at[pl.ds(s1_0 + c2a, E)],
                         rbuf_ref.at[pl.ds(c2a, E)], 4, py)
        sbuf_ref[pl.ds(s1_1, Q), :] = partial_batch(2 + (1 - x_b)).astype(
            jnp.bfloat16)
        p1b_1 = exchange(sbuf_ref.at[pl.ds(s1_1 + c1b, F)],
                         rbuf_ref.at[pl.ds(Q + c1b, F)], 1, px)
        p1b_2 = exchange(sbuf_ref.at[pl.ds(s1_1 + c1b + F, F)],
                         rbuf_ref.at[pl.ds(Q + c1b + F, F)], 3, px)
        p1b_3 = exchange(sbuf_ref.at[pl.ds(s1_1 + c2b, E)],
                         rbuf_ref.at[pl.ds(Q + c2b, E)], 5, px)

        acc_ref[pl.ds(k0, Q), :] = partial_batch(y_b)
        acc_ref[pl.ds(k1, Q), :] = partial_batch(2 + x_b)

        def fuse2(dst, a_off, r_off, n):
            sbuf_ref[pl.ds(dst, n), :] = (
                acc_ref[pl.ds(a_off, n), :]
                + rbuf_ref[pl.ds(r_off, n), :].astype(jnp.float32)
            ).astype(jnp.bfloat16)

        p1a_1.wait_recv()
        fuse2(s2_0, s2_0, c1a, F)
        p2a_1 = exchange(sbuf_ref.at[pl.ds(s2_0, F)],
                         rbuf_ref.at[pl.ds(2 * Q, F)], 6, px)
        p1b_1.wait_recv()
        fuse2(s2_1, s2_1, Q + c1b, F)
        p2b_1 = exchange(sbuf_ref.at[pl.ds(s2_1, F)],
                         rbuf_ref.at[pl.ds(2 * Q + E, F)], 7, py)
        p1a_2.wait_recv()
        fuse2(s2_0 + F, s2_0 + F, c1a + F, F)
        p2a_2 = exchange(sbuf_ref.at[pl.ds(s2_0 + F, F)],
                         rbuf_ref.at[pl.ds(2 * Q + F, F)], 8, px)
        p1b_2.wait_recv()
        fuse2(s2_1 + F, s2_1 + F, Q + c1b + F, F)
        p2b_2 = exchange(sbuf_ref.at[pl.ds(s2_1 + F, F)],
                         rbuf_ref.at[pl.ds(2 * Q + E + F, F)], 9, py)

        p1a_3.wait_recv()
        p1b_3.wait_recv()

        def outwrite(bat, seq, a_off, r1_off, r2_off):
            out_ref[pl.ds(bat, 1), pl.ds(seq, F), :] = (
                acc_ref[pl.ds(a_off, F), :]
                + rbuf_ref[pl.ds(r1_off, F), :].astype(jnp.float32)
                + rbuf_ref[pl.ds(r2_off, F), :].astype(jnp.float32)
            ).astype(jnp.bfloat16).reshape(1, F, N)

        p2b_1.wait_recv()
        outwrite(2 + x_b, y_b * E, k2_1, Q + c2b, 2 * Q + E)
        o3b1 = out_ref.at[pl.ds(2 + x_b, 1), pl.ds(y_b * E, F)]
        p3b_1 = exchange(o3b1, o3b1, 11, py)
        p2a_1.wait_recv()
        outwrite(y_b, x_b * E, k2_0, c2a, 2 * Q)
        o3a1 = out_ref.at[pl.ds(y_b, 1), pl.ds(x_b * E, F)]
        p3a_1 = exchange(o3a1, o3a1, 10, px)
        p2b_2.wait_recv()
        outwrite(2 + x_b, y_b * E + F, k2_1 + F, Q + c2b + F, 2 * Q + E + F)
        o3b2 = out_ref.at[pl.ds(2 + x_b, 1), pl.ds(y_b * E + F, F)]
        p3b_2 = exchange(o3b2, o3b2, 13, py)
        p2a_2.wait_recv()
        outwrite(y_b, x_b * E + F, k2_0 + F, c2a + F, 2 * Q + F)
        o3a2 = out_ref.at[pl.ds(y_b, 1), pl.ds(x_b * E + F, F)]
        p3a_2 = exchange(o3a2, o3a2, 12, px)

        o4a = out_ref.at[pl.ds(y_b, 1), pl.ds(x_b * E, E)]
        p4a_own = exchange(o4a, o4a, 14, py)
        o4b = out_ref.at[pl.ds(2 + x_b, 1), pl.ds(y_b * E, E)]
        p4b_own = exchange(o4b, o4b, 15, px)

        p3b_1.wait_recv()
        o4bg1 = out_ref.at[pl.ds(2 + x_b, 1), pl.ds((1 - y_b) * E, F)]
        p4b_g1 = exchange(o4bg1, o4bg1, 17, px)
        p3a_1.wait_recv()
        o4ag1 = out_ref.at[pl.ds(y_b, 1), pl.ds((1 - x_b) * E, F)]
        p4a_g1 = exchange(o4ag1, o4ag1, 16, py)
        p3b_2.wait_recv()
        o4bg2 = out_ref.at[pl.ds(2 + x_b, 1), pl.ds((1 - y_b) * E + F, F)]
        p4b_g2 = exchange(o4bg2, o4bg2, 19, px)
        p3a_2.wait_recv()
        o4ag2 = out_ref.at[pl.ds(y_b, 1), pl.ds((1 - x_b) * E + F, F)]
        p4a_g2 = exchange(o4ag2, o4ag2, 18, py)

        for r in (p4a_own, p4b_own, p4b_g1, p4a_g1, p4b_g2, p4a_g2):
            r.wait_recv()

        for r in rdmas:
            r.wait_send()

    return pl.pallas_call(
        body,
        out_shape=jax.ShapeDtypeStruct((B, S, N), jnp.bfloat16),
        in_specs=[
            pl.BlockSpec(memory_space=pltpu.VMEM),
            pl.BlockSpec(memory_space=pltpu.VMEM),
            pl.BlockSpec(memory_space=pltpu.VMEM),
        ],
        out_specs=pl.BlockSpec(memory_space=pltpu.VMEM),
        scratch_shapes=[
            pltpu.VMEM((M, N), jnp.float32),
            pltpu.VMEM((M, N), jnp.bfloat16),
            pltpu.VMEM((3 * M // 4, N), jnp.bfloat16),
            pltpu.SemaphoreType.DMA((20,)),
            pltpu.SemaphoreType.DMA((20,)),
        ],
        compiler_params=pltpu.CompilerParams(collective_id=0),
    )(x, k, Wp)


# device time: 48740 ns/iter; 3.3009x vs baseline; 1.0126x over previous
import jax
import jax.numpy as jnp
from jax import lax
from jax.experimental import pallas as pl
from jax.experimental.pallas import tpu as pltpu

N_DEV = 4


def kernel(x, k, Wp):
    B, S, C = x.shape
    K = k.shape[0]
    N = Wp.shape[1]
    M = B * S
    H = M // 2
    Q = H // 2
    E = Q // 2
    F = E // 2

    def body(x_hbm, k_hbm, w_hbm, out_ref, acc_ref, sbuf_ref, rbuf_ref,
             xb0_ref, xb1_ref, kw_ref, ww_ref,
             send_sems, recv_sems, dma_sems):
        my = lax.axis_index("i")
        x_b = my // 2
        y_b = (my // 2) ^ (my % 2)
        py = my ^ 1
        px = (N_DEV - 1) - my

        b_send0, b_send1 = 1 - y_b, 2 + (1 - x_b)
        b_kept0, b_kept1 = y_b, 2 + x_b
        cp_k = pltpu.make_async_copy(k_hbm, kw_ref, dma_sems.at[0])
        cp_k.start()
        cp_w = pltpu.make_async_copy(w_hbm, ww_ref, dma_sems.at[1])
        cp_w.start()
        cp_x0 = pltpu.make_async_copy(
            x_hbm.at[pl.ds(b_send0, 1)], xb0_ref, dma_sems.at[2])
        cp_x0.start()
        cp_x1 = pltpu.make_async_copy(
            x_hbm.at[pl.ds(b_send1, 1)], xb1_ref, dma_sems.at[3])
        cp_x1.start()

        barrier_sem = pltpu.get_barrier_semaphore()
        for nbr in (py, px):
            pl.semaphore_signal(
                barrier_sem, inc=1,
                device_id=(nbr,), device_id_type=pl.DeviceIdType.MESH,
            )

        cp_k.wait()
        cp_w.wait()
        kb = kw_ref[:, :].astype(jnp.bfloat16)
        w = ww_ref[:, :].astype(jnp.bfloat16)

        def partial_batch(buf_ref):
            xb = buf_ref[:, :, :].astype(jnp.bfloat16)
            conv = xb * kb[K - 1, :]
            for t in range(K - 1):
                shift = K - 1 - t
                shifted = jnp.concatenate(
                    [jnp.zeros((1, shift, C), dtype=xb.dtype),
                     xb[:, : S - shift, :]],
                    axis=1,
                )
                conv = conv + shifted * kb[t, :]
            cf = conv.astype(jnp.float32)
            a = cf * (1.0 / (1.0 + jnp.exp(-cf)))
            a2 = a.reshape(S, C).astype(jnp.bfloat16)
            return jnp.dot(a2, w, preferred_element_type=jnp.float32)

        rdmas = []

        def exchange(src_ref, dst_ref, sem_idx, partner):
            rdma = pltpu.make_async_remote_copy(
                src_ref=src_ref,
                dst_ref=dst_ref,
                send_sem=send_sems.at[sem_idx],
                recv_sem=recv_sems.at[sem_idx],
                device_id=(partner,),
                device_id_type=pl.DeviceIdType.MESH,
            )
            rdma.start()
            rdmas.append(rdma)
            return rdma

        k0 = y_b * Q
        k1 = H + x_b * Q
        s1_0 = (1 - y_b) * Q
        s1_1 = H + (1 - x_b) * Q
        s2_0 = k0 + (1 - x_b) * E
        s2_1 = k1 + (1 - y_b) * E
        k2_0 = k0 + x_b * E
        k2_1 = k1 + y_b * E

        c1a = (1 - x_b) * E
        c2a = x_b * E
        c1b = (1 - y_b) * E
        c2b = y_b * E

        cp_x0.wait()
        sbuf_ref[pl.ds(s1_0, Q), :] = partial_batch(xb0_ref).astype(
            jnp.bfloat16)
        pl.semaphore_wait(barrier_sem, 2)
        p1a_1 = exchange(sbuf_ref.at[pl.ds(s1_0 + c1a, F)],
                         rbuf_ref.at[pl.ds(c1a, F)], 0, py)
        p1a_2 = exchange(sbuf_ref.at[pl.ds(s1_0 + c1a + F, F)],
                         rbuf_ref.at[pl.ds(c1a + F, F)], 2, py)
        p1a_3 = exchange(sbuf_ref.at[pl.ds(s1_0 + c2a, E)],
                         rbuf_ref.at[pl.ds(c2a, E)], 4, py)
        cp_x1.wait()
        cp_x2 = pltpu.make_async_copy(
            x_hbm.at[pl.ds(b_kept0, 1)], xb0_ref, dma_sems.at[4])
        cp_x2.start()
        sbuf_ref[pl.ds(s1_1, Q), :] = partial_batch(xb1_ref).astype(
            jnp.bfloat16)
        p1b_1 = exchange(sbuf_ref.at[pl.ds(s1_1 + c1b, F)],
                         rbuf_ref.at[pl.ds(Q + c1b, F)], 1, px)
        p1b_2 = exchange(sbuf_ref.at[pl.ds(s1_1 + c1b + F, F)],
                         rbuf_ref.at[pl.ds(Q + c1b + F, F)], 3, px)
        p1b_3 = exchange(sbuf_ref.at[pl.ds(s1_1 + c2b, E)],
                         rbuf_ref.at[pl.ds(Q + c2b, E)], 5, px)
        cp_x3 = pltpu.make_async_copy(
            x_hbm.at[pl.ds(b_kept1, 1)], xb1_ref, dma_sems.at[5])

        cp_x2.wait()
        cp_x3.start()
        acc_ref[pl.ds(k0, Q), :] = partial_batch(xb0_ref)
        cp_x3.wait()
        acc_ref[pl.ds(k1, Q), :] = partial_batch(xb1_ref)

        def fuse2(dst, a_off, r_off, n):
            sbuf_ref[pl.ds(dst, n), :] = (
                acc_ref[pl.ds(a_off, n), :]
                + rbuf_ref[pl.ds(r_off, n), :].astype(jnp.float32)
            ).astype(jnp.bfloat16)

        p1a_1.wait_recv()
        fuse2(s2_0, s2_0, c1a, F)
        p2a_1 = exchange(sbuf_ref.at[pl.ds(s2_0, F)],
                         rbuf_ref.at[pl.ds(2 * Q, F)], 6, px)
        p1b_1.wait_recv()
        fuse2(s2_1, s2_1, Q + c1b, F)
        p2b_1 = exchange(sbuf_ref.at[pl.ds(s2_1, F)],
                         rbuf_ref.at[pl.ds(2 * Q + E, F)], 7, py)
        p1a_2.wait_recv()
        fuse2(s2_0 + F, s2_0 + F, c1a + F, F)
        p2a_2 = exchange(sbuf_ref.at[pl.ds(s2_0 + F, F)],
                         rbuf_ref.at[pl.ds(2 * Q + F, F)], 8, px)
        p1b_2.wait_recv()
        fuse2(s2_1 + F, s2_1 + F, Q + c1b + F, F)
        p2b_2 = exchange(sbuf_ref.at[pl.ds(s2_1 + F, F)],
                         rbuf_ref.at[pl.ds(2 * Q + E + F, F)], 9, py)

        p1a_3.wait_recv()
        p1b_3.wait_recv()

        def outwrite(bat, seq, a_off, r1_off, r2_off):
            out_ref[pl.ds(bat, 1), pl.ds(seq, F), :] = (
                acc_ref[pl.ds(a_off, F), :]
                + rbuf_ref[pl.ds(r1_off, F), :].astype(jnp.float32)
                + rbuf_ref[pl.ds(r2_off, F), :].astype(jnp.float32)
            ).astype(jnp.bfloat16).reshape(1, F, N)

        p2b_1.wait_recv()
        outwrite(2 + x_b, y_b * E, k2_1, Q + c2b, 2 * Q + E)
        o3b1 = out_ref.at[pl.ds(2 + x_b, 1), pl.ds(y_b * E, F)]
        p3b_1 = exchange(o3b1, o3b1, 11, py)
        p2a_1.wait_recv()
        outwrite(y_b, x_b * E, k2_0, c2a, 2 * Q)
        o3a1 = out_ref.at[pl.ds(y_b, 1), pl.ds(x_b * E, F)]
        p3a_1 = exchange(o3a1, o3a1, 10, px)
        p2b_2.wait_recv()
        outwrite(2 + x_b, y_b * E + F, k2_1 + F, Q + c2b + F, 2 * Q + E + F)
        o3b2 = out_ref.at[pl.ds(2 + x_b, 1), pl.ds(y_b * E + F, F)]
        p3b_2 = exchange(o3b2, o3b2, 13, py)
        p2a_2.wait_recv()
        outwrite(y_b, x_b * E + F, k2_0 + F, c2a + F, 2 * Q + F)
        o3a2 = out_ref.at[pl.ds(y_b, 1), pl.ds(x_b * E + F, F)]
        p3a_2 = exchange(o3a2, o3a2, 12, px)

        o4a = out_ref.at[pl.ds(y_b, 1), pl.ds(x_b * E, E)]
        p4a_own = exchange(o4a, o4a, 14, py)
        o4b = out_ref.at[pl.ds(2 + x_b, 1), pl.ds(y_b * E, E)]
        p4b_own = exchange(o4b, o4b, 15, px)

        p3b_1.wait_recv()
        o4bg1 = out_ref.at[pl.ds(2 + x_b, 1), pl.ds((1 - y_b) * E, F)]
        p4b_g1 = exchange(o4bg1, o4bg1, 17, px)
        p3a_1.wait_recv()
        o4ag1 = out_ref.at[pl.ds(y_b, 1), pl.ds((1 - x_b) * E, F)]
        p4a_g1 = exchange(o4ag1, o4ag1, 16, py)
        p3b_2.wait_recv()
        o4bg2 = out_ref.at[pl.ds(2 + x_b, 1), pl.ds((1 - y_b) * E + F, F)]
        p4b_g2 = exchange(o4bg2, o4bg2, 19, px)
        p3a_2.wait_recv()
        o4ag2 = out_ref.at[pl.ds(y_b, 1), pl.ds((1 - x_b) * E + F, F)]
        p4a_g2 = exchange(o4ag2, o4ag2, 18, py)

        for r in (p4a_own, p4b_own, p4b_g1, p4a_g1, p4b_g2, p4a_g2):
            r.wait_recv()

        for r in rdmas:
            r.wait_send()

    return pl.pallas_call(
        body,
        out_shape=jax.ShapeDtypeStruct((B, S, N), jnp.bfloat16),
        in_specs=[
            pl.BlockSpec(memory_space=pltpu.MemorySpace.HBM),
            pl.BlockSpec(memory_space=pltpu.MemorySpace.HBM),
            pl.BlockSpec(memory_space=pltpu.MemorySpace.HBM),
        ],
        out_specs=pl.BlockSpec(memory_space=pltpu.VMEM),
        scratch_shapes=[
            pltpu.VMEM((M, N), jnp.float32),
            pltpu.VMEM((M, N), jnp.bfloat16),
            pltpu.VMEM((3 * M // 4, N), jnp.bfloat16),
            pltpu.VMEM((1, S, C), jnp.float32),
            pltpu.VMEM((1, S, C), jnp.float32),
            pltpu.VMEM((K, C), jnp.float32),
            pltpu.VMEM((C, N), jnp.float32),
            pltpu.SemaphoreType.DMA((20,)),
            pltpu.SemaphoreType.DMA((20,)),
            pltpu.SemaphoreType.DMA((6,)),
        ],
        compiler_params=pltpu.CompilerParams(collective_id=0),
    )(x, k, Wp)
